# baseline (speedup 1.0000x reference)
import functools

import jax
import jax.numpy as jnp
from jax import lax
from jax.experimental import pallas as pl
from jax.experimental.pallas import tpu as pltpu
from jax.experimental.pallas import tpu_sc as plsc

N_NODES = 10000
N_EDGES = 320000
D = 128
N_GRAPHS = 64

NC = 2
NS = 16
NW = NC * NS

NP = 10240
BROWS = 320
CAP = 12288
CAPC = CAP // 128
SCAN_CHUNK = 3200
N_SCAN_CHUNKS = N_EDGES // SCAN_CHUNK
CHUNK = 128

_mesh = plsc.VectorSubcoreMesh(core_axis_name="c", subcore_axis_name="s")
_sc_params = pltpu.CompilerParams(needs_layout_passes=False)


@functools.partial(
    pl.kernel,
    mesh=_mesh,
    out_type=[
        jax.ShapeDtypeStruct((NW, CAP), jnp.int32),
        jax.ShapeDtypeStruct((NW, CAP), jnp.int32),
        jax.ShapeDtypeStruct((NW, 16), jnp.int32),
        jax.ShapeDtypeStruct((NW, BROWS), jnp.float32),
    ],
    scratch_types=[
        pltpu.VMEM((2, SCAN_CHUNK), jnp.int32),
        pltpu.VMEM((2, SCAN_CHUNK), jnp.int32),
        pltpu.VMEM((CAP,), jnp.int32),
        pltpu.VMEM((CAP,), jnp.int32),
        pltpu.VMEM((BROWS,), jnp.float32),
        pltpu.VMEM((16,), jnp.int32),
        pltpu.SemaphoreType.DMA,
        pltpu.SemaphoreType.DMA,
    ],
    compiler_params=_sc_params,
)
def _sc_partition(src_hbm, dst_hbm, srcf_hbm, dstf_hbm, cnt_hbm, deg_hbm,
                  srcb, dstb, srcout, dstout, degacc, cntv, sem0, sem1):
    cid = lax.axis_index("c")
    sid = lax.axis_index("s")
    wid = sid * NC + cid

    pad_src = jnp.full((16,), N_NODES, jnp.int32)
    zeros_i = jnp.zeros((16,), jnp.int32)
    zeros_f = jnp.zeros((16,), jnp.float32)

    def prefill(i, carry):
        srcout[pl.ds(i * 16, 16)] = pad_src
        dstout[pl.ds(i * 16, 16)] = zeros_i
        return carry

    lax.fori_loop(0, CAP // 16, prefill, 0)

    def zero_deg(i, carry):
        degacc[pl.ds(i * 16, 16)] = zeros_f
        return carry

    lax.fori_loop(0, BROWS // 16, zero_deg, 0)

    ones_f = jnp.ones((16,), jnp.float32)

    def process(buf, ptr_vec):
        def group(g, ptr):
            s = srcb[buf, pl.ds(g * 16, 16)]
            d = dstb[buf, pl.ds(g * 16, 16)]
            bkt = jnp.right_shift(d * 6554, 21)
            m = bkt == wid
            ld = d - wid * BROWS
            pos = plsc.cumsum(m.astype(jnp.int32))
            idx = ptr + pos - 1
            plsc.store_scatter(srcout, [idx], s, mask=m)
            plsc.store_scatter(dstout, [idx], ld, mask=m)
            plsc.addupdate_scatter(degacc, [ld], ones_f, mask=m)
            return ptr + plsc.all_reduce_population_count(m)

        return lax.fori_loop(0, SCAN_CHUNK // 16, group, ptr_vec, unroll=4)

    def start(chunk, buf, sem):
        pltpu.make_async_copy(
            src_hbm.at[pl.ds(chunk * SCAN_CHUNK, SCAN_CHUNK)],
            srcb.at[buf], sem).start()
        pltpu.make_async_copy(
            dst_hbm.at[pl.ds(chunk * SCAN_CHUNK, SCAN_CHUNK)],
            dstb.at[buf], sem).start()

    def wait(buf, sem):
        pltpu.make_async_copy(
            src_hbm.at[pl.ds(0, SCAN_CHUNK)], srcb.at[buf], sem).wait()
        pltpu.make_async_copy(
            dst_hbm.at[pl.ds(0, SCAN_CHUNK)], dstb.at[buf], sem).wait()

    start(0, 0, sem0)
    n_pairs = N_SCAN_CHUNKS // 2

    def pair(p, ptr_vec):
        wait(0, sem0)
        start(2 * p + 1, 1, sem1)
        ptr_vec = process(0, ptr_vec)
        wait(1, sem1)

        @pl.when(p + 1 < n_pairs)
        def _():
            start(2 * p + 2, 0, sem0)

        return process(1, ptr_vec)

    ptr_vec = lax.fori_loop(0, n_pairs, pair, jnp.zeros((16,), jnp.int32))

    cntv[...] = ptr_vec
    pltpu.sync_copy(srcout, srcf_hbm.at[wid])
    pltpu.sync_copy(dstout, dstf_hbm.at[wid])
    pltpu.sync_copy(cntv, cnt_hbm.at[wid])
    pltpu.sync_copy(degacc, deg_hbm.at[wid])


@functools.partial(
    pl.kernel,
    mesh=_mesh,
    out_type=jax.ShapeDtypeStruct((NP, D), jnp.float32),
    scratch_types=[
        pltpu.VMEM((CAPC, CHUNK), jnp.int32),
        pltpu.VMEM((CAPC, CHUNK), jnp.int32),
        pltpu.VMEM((CHUNK, D), jnp.float32),
        pltpu.VMEM((CHUNK, D), jnp.float32),
        pltpu.VMEM((BROWS, D), jnp.float32),
        pltpu.VMEM((16,), jnp.int32),
        pltpu.SemaphoreType.DMA,
        pltpu.SemaphoreType.DMA,
    ],
    compiler_params=_sc_params,
)
def _sc_layer(g_hbm, srcf_hbm, dstf_hbm, cnt_hbm, out_hbm,
              srcv, dstv, rows0, rows1, acc, cntv, sem0, sem1):
    cid = lax.axis_index("c")
    sid = lax.axis_index("s")
    wid = sid * NC + cid

    pltpu.sync_copy(srcf_hbm.at[wid], srcv)
    pltpu.sync_copy(dstf_hbm.at[wid], dstv)
    pltpu.sync_copy(cnt_hbm.at[wid], cntv)

    zeros_f = jnp.zeros((16,), jnp.float32)

    def zero_row(r, carry):
        for c in range(D // 16):
            acc[r, pl.ds(c * 16, 16)] = zeros_f
        return carry

    lax.fori_loop(0, BROWS, zero_row, 0)

    cnt = cntv[...][0]
    n_pairs = jnp.maximum((cnt + 255) >> 8, 1)

    def start(chunk, rows, sem):
        pltpu.make_async_copy(g_hbm.at[srcv.at[chunk]], rows, sem).start()

    def wait(rows, sem):
        pltpu.make_async_copy(g_hbm.at[srcv.at[0]], rows, sem).wait()

    e16 = lax.iota(jnp.int32, 16)

    def accumulate(rows, j):
        for grp in range(CHUNK // 16):
            ld16 = dstv[j, pl.ds(grp * 16, 16)]
            eg = e16 + grp * 16

            def col(c, carry):
                cv = jnp.full((16,), c, jnp.int32)
                vals = plsc.load_gather(rows, [eg, cv])
                plsc.addupdate_scatter(acc, [ld16, cv], vals)
                return carry

            lax.fori_loop(0, D, col, 0, unroll=8)

    start(0, rows0, sem0)

    def pair(q, carry):
        wait(rows0, sem0)
        start(2 * q + 1, rows1, sem1)
        accumulate(rows0, 2 * q)
        wait(rows1, sem1)

        @pl.when(q + 1 < n_pairs)
        def _():
            start(2 * q + 2, rows0, sem0)

        accumulate(rows1, 2 * q + 1)
        return carry

    lax.fori_loop(0, n_pairs, pair, 0)

    pltpu.sync_copy(acc, out_hbm.at[pl.ds(wid * BROWS, BROWS)])


def _dinv_from_deg(deg):
    rid = lax.broadcasted_iota(jnp.int32, (NP, 1), 0)
    return jnp.where(rid < N_NODES, lax.rsqrt(deg + 1.0), 0.0)


def _tc_first(x_ref, w_ref, deg_ref, g_ref):
    dinv = _dinv_from_deg(deg_ref[...])
    h = jnp.dot(x_ref[...], w_ref[...], preferred_element_type=jnp.float32)
    g_ref[...] = h * dinv


def _tc_mid(acc_ref, deg_ref, g_ref, b_ref, w_ref, out_ref):
    dinv = _dinv_from_deg(deg_ref[...])
    s = acc_ref[...] + g_ref[...]
    h = jnp.maximum(s * dinv + b_ref[...], 0.0)
    out_ref[...] = jnp.dot(h, w_ref[...],
                           preferred_element_type=jnp.float32) * dinv


def _tc_final(acc_ref, deg_ref, g_ref, b_ref, batch_ref, wlin_ref, blin_ref,
              out_ref):
    dinv = _dinv_from_deg(deg_ref[...])
    s = acc_ref[...] + g_ref[...]
    h = jnp.maximum(s * dinv + b_ref[...], 0.0)
    gid = lax.broadcasted_iota(jnp.int32, (N_GRAPHS, NP), 0)
    onehot = (gid == batch_ref[...]).astype(jnp.float32)
    summed = jnp.dot(onehot, h, preferred_element_type=jnp.float32)
    counts = jnp.sum(onehot, axis=1)[:, None]
    pooled = summed / jnp.maximum(counts, 1.0)
    out_ref[...] = (jnp.dot(pooled, wlin_ref[...],
                            preferred_element_type=jnp.float32)
                    + blin_ref[...])


_first_call = pl.pallas_call(
    _tc_first,
    out_shape=jax.ShapeDtypeStruct((NP, D), jnp.float32),
)

_mid_call = pl.pallas_call(
    _tc_mid,
    out_shape=jax.ShapeDtypeStruct((NP, D), jnp.float32),
)

_final_call = pl.pallas_call(
    _tc_final,
    out_shape=jax.ShapeDtypeStruct((N_GRAPHS, 1), jnp.float32),
)


def kernel(x, edge_index, batch, W1, b1, W2, b2, W3, b3, W_lin, b_lin):
    src = edge_index[0].astype(jnp.int32)
    dst = edge_index[1].astype(jnp.int32)
    x_pad = jnp.concatenate(
        [x, jnp.zeros((NP - N_NODES, D), jnp.float32)])
    batch_pad = jnp.concatenate(
        [batch.astype(jnp.int32),
         jnp.full((NP - N_NODES,), N_GRAPHS, jnp.int32)]).reshape(1, NP)

    srcf, dstf, cnt, deg = _sc_partition(src, dst)
    srcf = srcf.reshape(NW, CAPC, CHUNK)
    dstf = dstf.reshape(NW, CAPC, CHUNK)
    deg2 = deg.reshape(NP, 1)

    g = _first_call(x_pad, W1, deg2)
    for (b, w) in ((b1, W2), (b2, W3)):
        acc = _sc_layer(g, srcf, dstf, cnt)
        g = _mid_call(acc, deg2, g, b.reshape(1, D), w)
    acc = _sc_layer(g, srcf, dstf, cnt)
    out = _final_call(acc, deg2, g, b3.reshape(1, D), batch_pad,
                      W_lin, b_lin.reshape(1, 1))
    return out.reshape(-1)


# speedup vs baseline: 1.0008x; 1.0008x over previous
import functools

import jax
import jax.numpy as jnp
from jax import lax
from jax.experimental import pallas as pl
from jax.experimental.pallas import tpu as pltpu
from jax.experimental.pallas import tpu_sc as plsc

N_NODES = 10000
N_EDGES = 320000
D = 128
N_GRAPHS = 64

NC = 2
NS = 16
NW = NC * NS

NP = 10240
BROWS = 320
CAP = 12288
CAPC = CAP // 128
SCAN_CHUNK = 3200
N_SCAN_CHUNKS = N_EDGES // SCAN_CHUNK
CHUNK = 128

_mesh = plsc.VectorSubcoreMesh(core_axis_name="c", subcore_axis_name="s")
_sc_params = pltpu.CompilerParams(needs_layout_passes=False)


@functools.partial(
    pl.kernel,
    mesh=_mesh,
    out_type=[
        jax.ShapeDtypeStruct((NW, CAP), jnp.int32),
        jax.ShapeDtypeStruct((NW, CAP), jnp.int32),
        jax.ShapeDtypeStruct((NW, 16), jnp.int32),
        jax.ShapeDtypeStruct((NW, BROWS), jnp.float32),
    ],
    scratch_types=[
        pltpu.VMEM((2, SCAN_CHUNK), jnp.int32),
        pltpu.VMEM((2, SCAN_CHUNK), jnp.int32),
        pltpu.VMEM((CAP,), jnp.int32),
        pltpu.VMEM((CAP,), jnp.int32),
        pltpu.VMEM((BROWS,), jnp.float32),
        pltpu.VMEM((16,), jnp.int32),
        pltpu.SemaphoreType.DMA,
        pltpu.SemaphoreType.DMA,
    ],
    compiler_params=_sc_params,
)
def _sc_partition(src_hbm, dst_hbm, srcf_hbm, dstf_hbm, cnt_hbm, deg_hbm,
                  srcb, dstb, srcout, dstout, degacc, cntv, sem0, sem1):
    cid = lax.axis_index("c")
    sid = lax.axis_index("s")
    wid = sid * NC + cid

    pad_src = jnp.full((16,), N_NODES, jnp.int32)
    zeros_i = jnp.zeros((16,), jnp.int32)
    zeros_f = jnp.zeros((16,), jnp.float32)

    def prefill(i, carry):
        srcout[pl.ds(i * 16, 16)] = pad_src
        dstout[pl.ds(i * 16, 16)] = zeros_i
        return carry

    lax.fori_loop(0, CAP // 16, prefill, 0)

    def zero_deg(i, carry):
        degacc[pl.ds(i * 16, 16)] = zeros_f
        return carry

    lax.fori_loop(0, BROWS // 16, zero_deg, 0)

    ones_f = jnp.ones((16,), jnp.float32)
    one_i = jnp.ones((16,), jnp.int32)
    wid_vec = jnp.full((16,), wid, jnp.int32)
    wb_vec = wid_vec * BROWS

    def process(buf, ptr_vec):
        def group(g, ptr):
            s = srcb[buf, pl.ds(g * 16, 16)]
            d = dstb[buf, pl.ds(g * 16, 16)]
            bkt = jnp.right_shift(d * 6554, 21)
            m = bkt == wid_vec
            ld = d - wb_vec
            pos = plsc.cumsum(m.astype(jnp.int32))
            idx = ptr + pos - one_i
            plsc.store_scatter(srcout, [idx], s, mask=m)
            plsc.store_scatter(dstout, [idx], ld, mask=m)
            plsc.addupdate_scatter(degacc, [ld], ones_f, mask=m)
            return ptr + plsc.all_reduce_population_count(m)

        return lax.fori_loop(0, SCAN_CHUNK // 16, group, ptr_vec, unroll=4)

    def start(chunk, buf, sem):
        pltpu.make_async_copy(
            src_hbm.at[pl.ds(chunk * SCAN_CHUNK, SCAN_CHUNK)],
            srcb.at[buf], sem).start()
        pltpu.make_async_copy(
            dst_hbm.at[pl.ds(chunk * SCAN_CHUNK, SCAN_CHUNK)],
            dstb.at[buf], sem).start()

    def wait(buf, sem):
        pltpu.make_async_copy(
            src_hbm.at[pl.ds(0, SCAN_CHUNK)], srcb.at[buf], sem).wait()
        pltpu.make_async_copy(
            dst_hbm.at[pl.ds(0, SCAN_CHUNK)], dstb.at[buf], sem).wait()

    start(0, 0, sem0)
    n_pairs = N_SCAN_CHUNKS // 2

    def pair(p, ptr_vec):
        wait(0, sem0)
        start(2 * p + 1, 1, sem1)
        ptr_vec = process(0, ptr_vec)
        wait(1, sem1)

        @pl.when(p + 1 < n_pairs)
        def _():
            start(2 * p + 2, 0, sem0)

        return process(1, ptr_vec)

    ptr_vec = lax.fori_loop(0, n_pairs, pair, jnp.zeros((16,), jnp.int32))

    cntv[...] = ptr_vec
    pltpu.sync_copy(srcout, srcf_hbm.at[wid])
    pltpu.sync_copy(dstout, dstf_hbm.at[wid])
    pltpu.sync_copy(cntv, cnt_hbm.at[wid])
    pltpu.sync_copy(degacc, deg_hbm.at[wid])


@functools.partial(
    pl.kernel,
    mesh=_mesh,
    out_type=jax.ShapeDtypeStruct((NP, D), jnp.float32),
    scratch_types=[
        pltpu.VMEM((CAPC, CHUNK), jnp.int32),
        pltpu.VMEM((CAPC, CHUNK), jnp.int32),
        pltpu.VMEM((CHUNK, D), jnp.float32),
        pltpu.VMEM((CHUNK, D), jnp.float32),
        pltpu.VMEM((BROWS, D), jnp.float32),
        pltpu.VMEM((16,), jnp.int32),
        pltpu.SemaphoreType.DMA,
        pltpu.SemaphoreType.DMA,
    ],
    compiler_params=_sc_params,
)
def _sc_layer(g_hbm, srcf_hbm, dstf_hbm, cnt_hbm, out_hbm,
              srcv, dstv, rows0, rows1, acc, cntv, sem0, sem1):
    cid = lax.axis_index("c")
    sid = lax.axis_index("s")
    wid = sid * NC + cid

    pltpu.sync_copy(srcf_hbm.at[wid], srcv)
    pltpu.sync_copy(dstf_hbm.at[wid], dstv)
    pltpu.sync_copy(cnt_hbm.at[wid], cntv)

    zeros_f = jnp.zeros((16,), jnp.float32)

    def zero_row(r, carry):
        for c in range(D // 16):
            acc[r, pl.ds(c * 16, 16)] = zeros_f
        return carry

    lax.fori_loop(0, BROWS, zero_row, 0)

    cnt = cntv[...][0]
    n_pairs = jnp.maximum((cnt + 255) >> 8, 1)

    def start(chunk, rows, sem):
        pltpu.make_async_copy(g_hbm.at[srcv.at[chunk]], rows, sem).start()

    def wait(rows, sem):
        pltpu.make_async_copy(g_hbm.at[srcv.at[0]], rows, sem).wait()

    e16 = lax.iota(jnp.int32, 16)
    one_i = jnp.ones((16,), jnp.int32)
    cv0 = jnp.zeros((16,), jnp.int32)

    def accumulate(rows, j):
        for grp in range(CHUNK // 16):
            ld16 = dstv[j, pl.ds(grp * 16, 16)]
            eg = e16 + grp * 16

            def col(c, cv):
                vals = plsc.load_gather(rows, [eg, cv])
                plsc.addupdate_scatter(acc, [ld16, cv], vals)
                return cv + one_i

            lax.fori_loop(0, D, col, cv0, unroll=8)

    start(0, rows0, sem0)

    def pair(q, carry):
        wait(rows0, sem0)
        start(2 * q + 1, rows1, sem1)
        accumulate(rows0, 2 * q)
        wait(rows1, sem1)

        @pl.when(q + 1 < n_pairs)
        def _():
            start(2 * q + 2, rows0, sem0)

        accumulate(rows1, 2 * q + 1)
        return carry

    lax.fori_loop(0, n_pairs, pair, 0)

    pltpu.sync_copy(acc, out_hbm.at[pl.ds(wid * BROWS, BROWS)])


def _dinv_from_deg(deg):
    rid = lax.broadcasted_iota(jnp.int32, (NP, 1), 0)
    return jnp.where(rid < N_NODES, lax.rsqrt(deg + 1.0), 0.0)


def _tc_first(x_ref, w_ref, deg_ref, g_ref):
    dinv = _dinv_from_deg(deg_ref[...])
    h = jnp.dot(x_ref[...], w_ref[...], preferred_element_type=jnp.float32)
    g_ref[...] = h * dinv


def _tc_mid(acc_ref, deg_ref, g_ref, b_ref, w_ref, out_ref):
    dinv = _dinv_from_deg(deg_ref[...])
    s = acc_ref[...] + g_ref[...]
    h = jnp.maximum(s * dinv + b_ref[...], 0.0)
    out_ref[...] = jnp.dot(h, w_ref[...],
                           preferred_element_type=jnp.float32) * dinv


def _tc_final(acc_ref, deg_ref, g_ref, b_ref, batch_ref, wlin_ref, blin_ref,
              out_ref):
    dinv = _dinv_from_deg(deg_ref[...])
    s = acc_ref[...] + g_ref[...]
    h = jnp.maximum(s * dinv + b_ref[...], 0.0)
    gid = lax.broadcasted_iota(jnp.int32, (N_GRAPHS, NP), 0)
    onehot = (gid == batch_ref[...]).astype(jnp.float32)
    summed = jnp.dot(onehot, h, preferred_element_type=jnp.float32)
    counts = jnp.sum(onehot, axis=1)[:, None]
    pooled = summed / jnp.maximum(counts, 1.0)
    out_ref[...] = (jnp.dot(pooled, wlin_ref[...],
                            preferred_element_type=jnp.float32)
                    + blin_ref[...])


_first_call = pl.pallas_call(
    _tc_first,
    out_shape=jax.ShapeDtypeStruct((NP, D), jnp.float32),
)

_mid_call = pl.pallas_call(
    _tc_mid,
    out_shape=jax.ShapeDtypeStruct((NP, D), jnp.float32),
)

_final_call = pl.pallas_call(
    _tc_final,
    out_shape=jax.ShapeDtypeStruct((N_GRAPHS, 1), jnp.float32),
)


def kernel(x, edge_index, batch, W1, b1, W2, b2, W3, b3, W_lin, b_lin):
    src = edge_index[0].astype(jnp.int32)
    dst = edge_index[1].astype(jnp.int32)
    x_pad = jnp.concatenate(
        [x, jnp.zeros((NP - N_NODES, D), jnp.float32)])
    batch_pad = jnp.concatenate(
        [batch.astype(jnp.int32),
         jnp.full((NP - N_NODES,), N_GRAPHS, jnp.int32)]).reshape(1, NP)

    srcf, dstf, cnt, deg = _sc_partition(src, dst)
    srcf = srcf.reshape(NW, CAPC, CHUNK)
    dstf = dstf.reshape(NW, CAPC, CHUNK)
    deg2 = deg.reshape(NP, 1)

    g = _first_call(x_pad, W1, deg2)
    for (b, w) in ((b1, W2), (b2, W3)):
        acc = _sc_layer(g, srcf, dstf, cnt)
        g = _mid_call(acc, deg2, g, b.reshape(1, D), w)
    acc = _sc_layer(g, srcf, dstf, cnt)
    out = _final_call(acc, deg2, g, b3.reshape(1, D), batch_pad,
                      W_lin, b_lin.reshape(1, 1))
    return out.reshape(-1)


# speedup vs baseline: 3.4367x; 3.4339x over previous
import functools

import jax
import jax.numpy as jnp
from jax import lax
from jax.experimental import pallas as pl
from jax.experimental.pallas import tpu as pltpu
from jax.experimental.pallas import tpu_sc as plsc

N_NODES = 10000
N_EDGES = 320000
D = 128
N_GRAPHS = 64

NC = 2
NS = 16
NW = NC * NS

NP = 10240
BROWS = 320
CAP = 12288
CAPC = CAP // 128
SCAN_CHUNK = 3200
N_SCAN_CHUNKS = N_EDGES // SCAN_CHUNK
CHUNK = 128

_mesh = plsc.VectorSubcoreMesh(core_axis_name="c", subcore_axis_name="s")
_sc_params = pltpu.CompilerParams(needs_layout_passes=False)


@functools.partial(
    pl.kernel,
    mesh=_mesh,
    out_type=[
        jax.ShapeDtypeStruct((NW, CAP), jnp.int32),
        jax.ShapeDtypeStruct((NW, CAP), jnp.int32),
        jax.ShapeDtypeStruct((NW, 16), jnp.int32),
        jax.ShapeDtypeStruct((NW, BROWS), jnp.float32),
    ],
    scratch_types=[
        pltpu.VMEM((2, SCAN_CHUNK), jnp.int32),
        pltpu.VMEM((2, SCAN_CHUNK), jnp.int32),
        pltpu.VMEM((CAP,), jnp.int32),
        pltpu.VMEM((CAP,), jnp.int32),
        pltpu.VMEM((BROWS,), jnp.float32),
        pltpu.VMEM((16,), jnp.int32),
        pltpu.SemaphoreType.DMA,
        pltpu.SemaphoreType.DMA,
    ],
    compiler_params=_sc_params,
)
def _sc_partition(src_hbm, dst_hbm, srcf_hbm, dstf_hbm, cnt_hbm, deg_hbm,
                  srcb, dstb, srcout, dstout, degacc, cntv, sem0, sem1):
    cid = lax.axis_index("c")
    sid = lax.axis_index("s")
    wid = sid * NC + cid

    pad_src = jnp.full((16,), N_NODES, jnp.int32)
    zeros_i = jnp.zeros((16,), jnp.int32)
    zeros_f = jnp.zeros((16,), jnp.float32)

    def prefill(i, carry):
        srcout[pl.ds(i * 16, 16)] = pad_src
        dstout[pl.ds(i * 16, 16)] = zeros_i
        return carry

    lax.fori_loop(0, CAP // 16, prefill, 0)

    def zero_deg(i, carry):
        degacc[pl.ds(i * 16, 16)] = zeros_f
        return carry

    lax.fori_loop(0, BROWS // 16, zero_deg, 0)

    ones_f = jnp.ones((16,), jnp.float32)
    one_i = jnp.ones((16,), jnp.int32)
    wid_vec = jnp.full((16,), wid, jnp.int32)
    wb_vec = wid_vec * BROWS

    def process(buf, ptr_vec):
        def group(g, ptr):
            s = srcb[buf, pl.ds(g * 16, 16)]
            d = dstb[buf, pl.ds(g * 16, 16)]
            bkt = jnp.right_shift(d * 6554, 21)
            m = bkt == wid_vec
            ld = d - wb_vec
            pos = plsc.cumsum(m.astype(jnp.int32))
            idx = ptr + pos - one_i
            plsc.store_scatter(srcout, [idx], s, mask=m)
            plsc.store_scatter(dstout, [idx], ld, mask=m)
            plsc.addupdate_scatter(degacc, [ld], ones_f, mask=m)
            return ptr + plsc.all_reduce_population_count(m)

        return lax.fori_loop(0, SCAN_CHUNK // 16, group, ptr_vec, unroll=4)

    def start(chunk, buf, sem):
        pltpu.make_async_copy(
            src_hbm.at[pl.ds(chunk * SCAN_CHUNK, SCAN_CHUNK)],
            srcb.at[buf], sem).start()
        pltpu.make_async_copy(
            dst_hbm.at[pl.ds(chunk * SCAN_CHUNK, SCAN_CHUNK)],
            dstb.at[buf], sem).start()

    def wait(buf, sem):
        pltpu.make_async_copy(
            src_hbm.at[pl.ds(0, SCAN_CHUNK)], srcb.at[buf], sem).wait()
        pltpu.make_async_copy(
            dst_hbm.at[pl.ds(0, SCAN_CHUNK)], dstb.at[buf], sem).wait()

    start(0, 0, sem0)
    n_pairs = N_SCAN_CHUNKS // 2

    def pair(p, ptr_vec):
        wait(0, sem0)
        start(2 * p + 1, 1, sem1)
        ptr_vec = process(0, ptr_vec)
        wait(1, sem1)

        @pl.when(p + 1 < n_pairs)
        def _():
            start(2 * p + 2, 0, sem0)

        return process(1, ptr_vec)

    ptr_vec = lax.fori_loop(0, n_pairs, pair, jnp.zeros((16,), jnp.int32))

    cntv[...] = ptr_vec
    pltpu.sync_copy(srcout, srcf_hbm.at[wid])
    pltpu.sync_copy(dstout, dstf_hbm.at[wid])
    pltpu.sync_copy(cntv, cnt_hbm.at[wid])
    pltpu.sync_copy(degacc, deg_hbm.at[wid])


@functools.partial(
    pl.kernel,
    mesh=_mesh,
    out_type=jax.ShapeDtypeStruct((NP, D), jnp.float32),
    scratch_types=[
        pltpu.VMEM((CAPC, CHUNK), jnp.int32),
        pltpu.VMEM((CAPC, CHUNK), jnp.int32),
        pltpu.VMEM((CHUNK, D), jnp.float32),
        pltpu.VMEM((CHUNK, D), jnp.float32),
        pltpu.VMEM((BROWS, D), jnp.float32),
        pltpu.VMEM((16,), jnp.int32),
        pltpu.SemaphoreType.DMA,
        pltpu.SemaphoreType.DMA,
    ],
    compiler_params=_sc_params,
)
def _sc_layer(g_hbm, srcf_hbm, dstf_hbm, cnt_hbm, out_hbm,
              srcv, dstv, rows0, rows1, acc, cntv, sem0, sem1):
    cid = lax.axis_index("c")
    sid = lax.axis_index("s")
    wid = sid * NC + cid

    pltpu.sync_copy(srcf_hbm.at[wid], srcv)
    pltpu.sync_copy(dstf_hbm.at[wid], dstv)
    pltpu.sync_copy(cnt_hbm.at[wid], cntv)

    zeros_f = jnp.zeros((16,), jnp.float32)

    def zero_row(r, carry):
        for c in range(D // 16):
            acc[r, pl.ds(c * 16, 16)] = zeros_f
        return carry

    lax.fori_loop(0, BROWS, zero_row, 0)

    cnt = cntv[...][0]
    n_pairs = jnp.maximum((cnt + 255) >> 8, 1)

    def start(chunk, rows, sem):
        pltpu.make_async_copy(g_hbm.at[srcv.at[chunk]], rows, sem).start()

    def wait(rows, sem):
        pltpu.make_async_copy(g_hbm.at[srcv.at[0]], rows, sem).wait()

    def accumulate(rows, j):
        def grp_body(grp, carry):
            ld16 = dstv[j, pl.ds(grp * 16, 16)]
            base = grp * 16
            for e in range(16):
                lde = ld16[e]
                for c in range(D // 16):
                    plsc.addupdate(acc.at[lde, pl.ds(c * 16, 16)],
                                   rows[base + e, pl.ds(c * 16, 16)])
            return carry

        lax.fori_loop(0, CHUNK // 16, grp_body, 0)

    start(0, rows0, sem0)

    def pair(q, carry):
        wait(rows0, sem0)
        start(2 * q + 1, rows1, sem1)
        accumulate(rows0, 2 * q)
        wait(rows1, sem1)

        @pl.when(q + 1 < n_pairs)
        def _():
            start(2 * q + 2, rows0, sem0)

        accumulate(rows1, 2 * q + 1)
        return carry

    lax.fori_loop(0, n_pairs, pair, 0)

    pltpu.sync_copy(acc, out_hbm.at[pl.ds(wid * BROWS, BROWS)])


def _dinv_from_deg(deg):
    rid = lax.broadcasted_iota(jnp.int32, (NP, 1), 0)
    return jnp.where(rid < N_NODES, lax.rsqrt(deg + 1.0), 0.0)


def _tc_first(x_ref, w_ref, deg_ref, g_ref):
    dinv = _dinv_from_deg(deg_ref[...])
    h = jnp.dot(x_ref[...], w_ref[...], preferred_element_type=jnp.float32)
    g_ref[...] = h * dinv


def _tc_mid(acc_ref, deg_ref, g_ref, b_ref, w_ref, out_ref):
    dinv = _dinv_from_deg(deg_ref[...])
    s = acc_ref[...] + g_ref[...]
    h = jnp.maximum(s * dinv + b_ref[...], 0.0)
    out_ref[...] = jnp.dot(h, w_ref[...],
                           preferred_element_type=jnp.float32) * dinv


def _tc_final(acc_ref, deg_ref, g_ref, b_ref, batch_ref, wlin_ref, blin_ref,
              out_ref):
    dinv = _dinv_from_deg(deg_ref[...])
    s = acc_ref[...] + g_ref[...]
    h = jnp.maximum(s * dinv + b_ref[...], 0.0)
    gid = lax.broadcasted_iota(jnp.int32, (N_GRAPHS, NP), 0)
    onehot = (gid == batch_ref[...]).astype(jnp.float32)
    summed = jnp.dot(onehot, h, preferred_element_type=jnp.float32)
    counts = jnp.sum(onehot, axis=1)[:, None]
    pooled = summed / jnp.maximum(counts, 1.0)
    out_ref[...] = (jnp.dot(pooled, wlin_ref[...],
                            preferred_element_type=jnp.float32)
                    + blin_ref[...])


_first_call = pl.pallas_call(
    _tc_first,
    out_shape=jax.ShapeDtypeStruct((NP, D), jnp.float32),
)

_mid_call = pl.pallas_call(
    _tc_mid,
    out_shape=jax.ShapeDtypeStruct((NP, D), jnp.float32),
)

_final_call = pl.pallas_call(
    _tc_final,
    out_shape=jax.ShapeDtypeStruct((N_GRAPHS, 1), jnp.float32),
)


def kernel(x, edge_index, batch, W1, b1, W2, b2, W3, b3, W_lin, b_lin):
    src = edge_index[0].astype(jnp.int32)
    dst = edge_index[1].astype(jnp.int32)
    x_pad = jnp.concatenate(
        [x, jnp.zeros((NP - N_NODES, D), jnp.float32)])
    batch_pad = jnp.concatenate(
        [batch.astype(jnp.int32),
         jnp.full((NP - N_NODES,), N_GRAPHS, jnp.int32)]).reshape(1, NP)

    srcf, dstf, cnt, deg = _sc_partition(src, dst)
    srcf = srcf.reshape(NW, CAPC, CHUNK)
    dstf = dstf.reshape(NW, CAPC, CHUNK)
    deg2 = deg.reshape(NP, 1)

    g = _first_call(x_pad, W1, deg2)
    for (b, w) in ((b1, W2), (b2, W3)):
        acc = _sc_layer(g, srcf, dstf, cnt)
        g = _mid_call(acc, deg2, g, b.reshape(1, D), w)
    acc = _sc_layer(g, srcf, dstf, cnt)
    out = _final_call(acc, deg2, g, b3.reshape(1, D), batch_pad,
                      W_lin, b_lin.reshape(1, 1))
    return out.reshape(-1)


# speedup vs baseline: 5.0759x; 1.4770x over previous
import functools

import jax
import jax.numpy as jnp
from jax import lax
from jax.experimental import pallas as pl
from jax.experimental.pallas import tpu as pltpu
from jax.experimental.pallas import tpu_sc as plsc

N_NODES = 10000
N_EDGES = 320000
D = 128
N_GRAPHS = 64

NC = 2
NS = 16
NW = NC * NS

NP = 10240
BROWS = 320
CAP = 12288
CAPC = CAP // 128
SCAN_CHUNK = 3200
N_SCAN_CHUNKS = N_EDGES // SCAN_CHUNK
CHUNK = 128

_mesh = plsc.VectorSubcoreMesh(core_axis_name="c", subcore_axis_name="s")
_sc_params = pltpu.CompilerParams(needs_layout_passes=False)


@functools.partial(
    pl.kernel,
    mesh=_mesh,
    out_type=[
        jax.ShapeDtypeStruct((NW, CAP), jnp.int32),
        jax.ShapeDtypeStruct((NW, CAP), jnp.int32),
        jax.ShapeDtypeStruct((NW, 16), jnp.int32),
        jax.ShapeDtypeStruct((NW, BROWS), jnp.float32),
    ],
    scratch_types=[
        pltpu.VMEM((2, SCAN_CHUNK), jnp.int32),
        pltpu.VMEM((2, SCAN_CHUNK), jnp.int32),
        pltpu.VMEM((CAP,), jnp.int32),
        pltpu.VMEM((CAP,), jnp.int32),
        pltpu.VMEM((BROWS,), jnp.float32),
        pltpu.VMEM((16,), jnp.int32),
        pltpu.SemaphoreType.DMA,
        pltpu.SemaphoreType.DMA,
    ],
    compiler_params=_sc_params,
)
def _sc_partition(src_hbm, dst_hbm, srcf_hbm, dstf_hbm, cnt_hbm, deg_hbm,
                  srcb, dstb, srcout, dstout, degacc, cntv, sem0, sem1):
    cid = lax.axis_index("c")
    sid = lax.axis_index("s")
    wid = sid * NC + cid

    pad_src = jnp.full((16,), N_NODES, jnp.int32)
    zeros_i = jnp.zeros((16,), jnp.int32)
    zeros_f = jnp.zeros((16,), jnp.float32)

    def prefill(i, carry):
        srcout[pl.ds(i * 16, 16)] = pad_src
        dstout[pl.ds(i * 16, 16)] = zeros_i
        return carry

    lax.fori_loop(0, CAP // 16, prefill, 0)

    def zero_deg(i, carry):
        degacc[pl.ds(i * 16, 16)] = zeros_f
        return carry

    lax.fori_loop(0, BROWS // 16, zero_deg, 0)

    ones_f = jnp.ones((16,), jnp.float32)
    one_i = jnp.ones((16,), jnp.int32)
    wid_vec = jnp.full((16,), wid, jnp.int32)
    wb_vec = wid_vec * BROWS

    def process(buf, ptr_vec):
        def group(g, ptr):
            s = srcb[buf, pl.ds(g * 16, 16)]
            d = dstb[buf, pl.ds(g * 16, 16)]
            bkt = jnp.right_shift(d * 6554, 21)
            m = bkt == wid_vec
            ld = d - wb_vec
            pos = plsc.cumsum(m.astype(jnp.int32))
            idx = ptr + pos - one_i
            plsc.store_scatter(srcout, [idx], s, mask=m)
            plsc.store_scatter(dstout, [idx], ld, mask=m)
            plsc.addupdate_scatter(degacc, [ld], ones_f, mask=m)
            return ptr + plsc.all_reduce_population_count(m)

        return lax.fori_loop(0, SCAN_CHUNK // 16, group, ptr_vec, unroll=8)

    def start(chunk, buf, sem):
        pltpu.make_async_copy(
            src_hbm.at[pl.ds(chunk * SCAN_CHUNK, SCAN_CHUNK)],
            srcb.at[buf], sem).start()
        pltpu.make_async_copy(
            dst_hbm.at[pl.ds(chunk * SCAN_CHUNK, SCAN_CHUNK)],
            dstb.at[buf], sem).start()

    def wait(buf, sem):
        pltpu.make_async_copy(
            src_hbm.at[pl.ds(0, SCAN_CHUNK)], srcb.at[buf], sem).wait()
        pltpu.make_async_copy(
            dst_hbm.at[pl.ds(0, SCAN_CHUNK)], dstb.at[buf], sem).wait()

    start(0, 0, sem0)
    n_pairs = N_SCAN_CHUNKS // 2

    def pair(p, ptr_vec):
        wait(0, sem0)
        start(2 * p + 1, 1, sem1)
        ptr_vec = process(0, ptr_vec)
        wait(1, sem1)

        @pl.when(p + 1 < n_pairs)
        def _():
            start(2 * p + 2, 0, sem0)

        return process(1, ptr_vec)

    ptr_vec = lax.fori_loop(0, n_pairs, pair, jnp.zeros((16,), jnp.int32))

    cntv[...] = ptr_vec
    pltpu.sync_copy(srcout, srcf_hbm.at[wid])
    pltpu.sync_copy(dstout, dstf_hbm.at[wid])
    pltpu.sync_copy(cntv, cnt_hbm.at[wid])
    pltpu.sync_copy(degacc, deg_hbm.at[wid])


@functools.partial(
    pl.kernel,
    mesh=_mesh,
    out_type=jax.ShapeDtypeStruct((NP, D), jnp.float32),
    scratch_types=[
        pltpu.VMEM((CAPC, CHUNK), jnp.int32),
        pltpu.VMEM((CAPC, CHUNK), jnp.int32),
        pltpu.VMEM((CHUNK, D), jnp.float32),
        pltpu.VMEM((CHUNK, D), jnp.float32),
        pltpu.VMEM((BROWS, D), jnp.float32),
        pltpu.VMEM((16,), jnp.int32),
        pltpu.SemaphoreType.DMA,
        pltpu.SemaphoreType.DMA,
    ],
    compiler_params=_sc_params,
)
def _sc_layer(g_hbm, srcf_hbm, dstf_hbm, cnt_hbm, out_hbm,
              srcv, dstv, rows0, rows1, acc, cntv, sem0, sem1):
    cid = lax.axis_index("c")
    sid = lax.axis_index("s")
    wid = sid * NC + cid

    pltpu.sync_copy(srcf_hbm.at[wid], srcv)
    pltpu.sync_copy(dstf_hbm.at[wid], dstv)
    pltpu.sync_copy(cnt_hbm.at[wid], cntv)

    zeros_f = jnp.zeros((16,), jnp.float32)

    def zero_row(r, carry):
        for c in range(D // 16):
            acc[r, pl.ds(c * 16, 16)] = zeros_f
        return carry

    lax.fori_loop(0, BROWS, zero_row, 0)

    cnt = cntv[...][0]
    n_pairs = jnp.maximum((cnt + 255) >> 8, 1)

    def start(chunk, rows, sem):
        pltpu.make_async_copy(g_hbm.at[srcv.at[chunk]], rows, sem).start()

    def wait(rows, sem):
        pltpu.make_async_copy(g_hbm.at[srcv.at[0]], rows, sem).wait()

    def accumulate(rows, j):
        def grp_body(grp, carry):
            ld16 = dstv[j, pl.ds(grp * 16, 16)]
            base = grp * 16
            for e in range(16):
                lde = ld16[e]
                vals = [rows[base + e, pl.ds(c * 16, 16)]
                        for c in range(D // 16)]
                for c in range(D // 16):
                    plsc.addupdate(acc.at[lde, pl.ds(c * 16, 16)], vals[c])
            return carry

        lax.fori_loop(0, CHUNK // 16, grp_body, 0)

    start(0, rows0, sem0)

    def pair(q, carry):
        wait(rows0, sem0)
        start(2 * q + 1, rows1, sem1)
        accumulate(rows0, 2 * q)
        wait(rows1, sem1)

        @pl.when(q + 1 < n_pairs)
        def _():
            start(2 * q + 2, rows0, sem0)

        accumulate(rows1, 2 * q + 1)
        return carry

    lax.fori_loop(0, n_pairs, pair, 0)

    pltpu.sync_copy(acc, out_hbm.at[pl.ds(wid * BROWS, BROWS)])


def _dinv_from_deg(deg):
    rid = lax.broadcasted_iota(jnp.int32, (NP, 1), 0)
    return jnp.where(rid < N_NODES, lax.rsqrt(deg + 1.0), 0.0)


def _tc_first(x_ref, w_ref, deg_ref, g_ref):
    dinv = _dinv_from_deg(deg_ref[...])
    h = jnp.dot(x_ref[...], w_ref[...], preferred_element_type=jnp.float32)
    g_ref[...] = h * dinv


def _tc_mid(acc_ref, deg_ref, g_ref, b_ref, w_ref, out_ref):
    dinv = _dinv_from_deg(deg_ref[...])
    s = acc_ref[...] + g_ref[...]
    h = jnp.maximum(s * dinv + b_ref[...], 0.0)
    out_ref[...] = jnp.dot(h, w_ref[...],
                           preferred_element_type=jnp.float32) * dinv


def _tc_final(acc_ref, deg_ref, g_ref, b_ref, batch_ref, wlin_ref, blin_ref,
              out_ref):
    dinv = _dinv_from_deg(deg_ref[...])
    s = acc_ref[...] + g_ref[...]
    h = jnp.maximum(s * dinv + b_ref[...], 0.0)
    gid = lax.broadcasted_iota(jnp.int32, (N_GRAPHS, NP), 0)
    onehot = (gid == batch_ref[...]).astype(jnp.float32)
    summed = jnp.dot(onehot, h, preferred_element_type=jnp.float32)
    counts = jnp.sum(onehot, axis=1)[:, None]
    pooled = summed / jnp.maximum(counts, 1.0)
    out_ref[...] = (jnp.dot(pooled, wlin_ref[...],
                            preferred_element_type=jnp.float32)
                    + blin_ref[...])


_first_call = pl.pallas_call(
    _tc_first,
    out_shape=jax.ShapeDtypeStruct((NP, D), jnp.float32),
)

_mid_call = pl.pallas_call(
    _tc_mid,
    out_shape=jax.ShapeDtypeStruct((NP, D), jnp.float32),
)

_final_call = pl.pallas_call(
    _tc_final,
    out_shape=jax.ShapeDtypeStruct((N_GRAPHS, 1), jnp.float32),
)


def kernel(x, edge_index, batch, W1, b1, W2, b2, W3, b3, W_lin, b_lin):
    src = edge_index[0].astype(jnp.int32)
    dst = edge_index[1].astype(jnp.int32)
    x_pad = jnp.concatenate(
        [x, jnp.zeros((NP - N_NODES, D), jnp.float32)])
    batch_pad = jnp.concatenate(
        [batch.astype(jnp.int32),
         jnp.full((NP - N_NODES,), N_GRAPHS, jnp.int32)]).reshape(1, NP)

    srcf, dstf, cnt, deg = _sc_partition(src, dst)
    srcf = srcf.reshape(NW, CAPC, CHUNK)
    dstf = dstf.reshape(NW, CAPC, CHUNK)
    deg2 = deg.reshape(NP, 1)

    g = _first_call(x_pad, W1, deg2)
    for (b, w) in ((b1, W2), (b2, W3)):
        acc = _sc_layer(g, srcf, dstf, cnt)
        g = _mid_call(acc, deg2, g, b.reshape(1, D), w)
    acc = _sc_layer(g, srcf, dstf, cnt)
    out = _final_call(acc, deg2, g, b3.reshape(1, D), batch_pad,
                      W_lin, b_lin.reshape(1, 1))
    return out.reshape(-1)
